# Initial kernel scaffold; baseline (speedup 1.0000x reference)
import jax
import jax.numpy as jnp
from jax.experimental import pallas as pl


def kernel(X, A, W, lin_w, lin_b):
    raise NotImplementedError("write your pallas kernel here")


# speedup vs baseline: 1.6370x; 1.6370x over previous
import functools

import jax
import jax.numpy as jnp
from jax.experimental import pallas as pl
from jax.experimental.pallas import tpu as pltpu

_BB = 8
_G = 4


def _diff_conv_body(n, din, dout, k_hops, bb, g_unroll,
                    a_ref, x_ref, ones_ref, w_ref, lw_ref, lb_ref, o_ref):
    bbn = bb * n
    f32 = jnp.float32
    bf16 = jnp.bfloat16
    ones_bd = ones_ref[...]
    wmat = w_ref[:, 0:n]
    lw = lw_ref[...]
    lb = lb_ref[0:1, :]

    for g in range(g_unroll):
        a = a_ref[g * n:(g + 1) * n, :]
        x = x_ref[g * bbn:(g + 1) * bbn, :]

        rs = jnp.dot(a, ones_bd, preferred_element_type=f32)
        trans = a * pl.reciprocal(rs + 1e-12, approx=False)

        logits = jnp.dot(wmat, trans, preferred_element_type=f32)

        e = jnp.exp(logits)
        den = jnp.dot(e, ones_bd, preferred_element_type=f32)
        t = e * pl.reciprocal(den, approx=False)

        t_bd = jnp.concatenate([t] * bb, axis=0) * ones_bd
        tb = t_bd.astype(bf16)

        xb = x.astype(bf16)
        x1 = jnp.dot(tb, xb, preferred_element_type=f32)
        x1b = x1.astype(bf16)
        x2b = jnp.dot(tb, x1b, preferred_element_type=f32).astype(bf16)
        xcat = jnp.concatenate([xb, x1b, x2b], axis=1)
        acc = jnp.dot(xcat, lw, preferred_element_type=f32)
        acc = (acc + lb) * (1.0 / k_hops)
        o_ref[g * bbn:(g + 1) * bbn, :] = jnp.maximum(acc, 0.0)


def kernel(X, A, W, lin_w, lin_b):
    f32 = jnp.float32
    b, n, din = X.shape
    k_hops, _, dout = lin_w.shape
    bb = _BB
    bbn = bb * n
    c = b // bb
    g_unroll = _G
    s = c // g_unroll

    X2 = X.reshape(b * n, din).astype(f32)

    A_cmp = (A.astype(f32)
             .reshape(s * g_unroll, bb, n, n)
             .transpose(0, 2, 1, 3)
             .reshape(c * n, bbn))

    ones_bd = jnp.kron(jnp.eye(bb, dtype=f32), jnp.ones((n, n), f32))

    w_pad = jnp.pad(W.reshape(n, n).astype(f32), ((0, 0), (0, bbn - n)))
    lw = lin_w.reshape(k_hops * din, dout).astype(jnp.bfloat16)
    lb = jnp.pad(jnp.sum(lin_b.astype(f32), axis=0, keepdims=True),
                 ((0, 7), (0, 0)))

    body = functools.partial(_diff_conv_body, n, din, dout, k_hops, bb,
                             g_unroll)
    out2 = pl.pallas_call(
        body,
        out_shape=jax.ShapeDtypeStruct((b * n, dout), f32),
        grid=(s,),
        in_specs=[
            pl.BlockSpec((g_unroll * n, bbn), lambda i: (i, 0)),
            pl.BlockSpec((g_unroll * bbn, din), lambda i: (i, 0)),
            pl.BlockSpec((bbn, bbn), lambda i: (0, 0)),
            pl.BlockSpec((n, bbn), lambda i: (0, 0)),
            pl.BlockSpec((k_hops * din, dout), lambda i: (0, 0)),
            pl.BlockSpec((8, dout), lambda i: (0, 0)),
        ],
        out_specs=pl.BlockSpec((g_unroll * bbn, dout), lambda i: (i, 0)),
        compiler_params=pltpu.CompilerParams(
            dimension_semantics=("parallel",)),
    )(A_cmp, X2, ones_bd, w_pad, lw, lb)
    return out2.reshape(b, n, dout)


# speedup vs baseline: 2.6257x; 1.6040x over previous
import functools

import jax
import jax.numpy as jnp
from jax.experimental import pallas as pl
from jax.experimental.pallas import tpu as pltpu

_BB = 8
_G = 8


def _diff_conv_body(n, din, dout, k_hops, bb, g_unroll,
                    a_ref, x_ref, ones_ref, wbd_ref, lw_ref, lb_ref, o_ref):
    bbn = bb * n
    f32 = jnp.float32
    bf16 = jnp.bfloat16
    ones_bd = ones_ref[...]
    wbd = wbd_ref[...]
    lw = lw_ref[...]
    lb = lb_ref[0:1, :]

    a = a_ref[...]
    rs = jnp.dot(a, ones_bd, preferred_element_type=f32)
    trans = (a.astype(f32) * pl.reciprocal(rs + 1e-12, approx=False)
             ).astype(bf16)
    logits = jnp.dot(wbd, trans, preferred_element_type=f32)
    e = jnp.exp(logits)
    den = jnp.dot(e.astype(bf16), ones_bd, preferred_element_type=f32)
    t = (e * pl.reciprocal(den, approx=False)).astype(bf16)

    xb = x_ref[...]
    for q in range(g_unroll):
        tq = t[q * n:(q + 1) * n, :]
        t_bd = jnp.concatenate([tq] * bb, axis=0) * ones_bd
        xq = xb[q * bbn:(q + 1) * bbn, :]
        x1 = jnp.dot(t_bd, xq, preferred_element_type=f32)
        x1b = x1.astype(bf16)
        x2b = jnp.dot(t_bd, x1b, preferred_element_type=f32).astype(bf16)
        xcat = jnp.concatenate([xq, x1b, x2b], axis=1)
        acc = jnp.dot(xcat, lw, preferred_element_type=f32)
        acc = (acc + lb) * (1.0 / k_hops)
        o_ref[q * bbn:(q + 1) * bbn, :] = jnp.maximum(acc, 0.0)


def kernel(X, A, W, lin_w, lin_b):
    f32 = jnp.float32
    bf16 = jnp.bfloat16
    b, n, din = X.shape
    k_hops, _, dout = lin_w.shape
    bb = _BB
    bbn = bb * n
    c = b // bb
    g_unroll = _G
    s = c // g_unroll

    X2 = X.reshape(b * n, din).astype(bf16)

    A_cmp = (A.astype(bf16)
             .reshape(c, bb, n, n)
             .transpose(0, 2, 1, 3)
             .reshape(c * n, bbn))

    ones_bd = jnp.kron(jnp.eye(bb, dtype=bf16), jnp.ones((n, n), bf16))
    w_bd = jnp.kron(jnp.eye(g_unroll, dtype=f32),
                    W.reshape(n, n).astype(f32)).astype(bf16)

    lw = lin_w.reshape(k_hops * din, dout).astype(bf16)
    lb = jnp.pad(jnp.sum(lin_b.astype(f32), axis=0, keepdims=True),
                 ((0, 7), (0, 0)))

    body = functools.partial(_diff_conv_body, n, din, dout, k_hops, bb,
                             g_unroll)
    out2 = pl.pallas_call(
        body,
        out_shape=jax.ShapeDtypeStruct((b * n, dout), f32),
        grid=(s,),
        in_specs=[
            pl.BlockSpec((g_unroll * n, bbn), lambda i: (i, 0)),
            pl.BlockSpec((g_unroll * bbn, din), lambda i: (i, 0)),
            pl.BlockSpec((bbn, bbn), lambda i: (0, 0)),
            pl.BlockSpec((g_unroll * n, g_unroll * n), lambda i: (0, 0)),
            pl.BlockSpec((k_hops * din, dout), lambda i: (0, 0)),
            pl.BlockSpec((8, dout), lambda i: (0, 0)),
        ],
        out_specs=pl.BlockSpec((g_unroll * bbn, dout), lambda i: (i, 0)),
        compiler_params=pltpu.CompilerParams(
            dimension_semantics=("parallel",)),
    )(A_cmp, X2, ones_bd, w_bd, lw, lb)
    return out2.reshape(b, n, dout)


# speedup vs baseline: 4.1725x; 1.5891x over previous
import functools

import jax
import jax.numpy as jnp
from jax.experimental import pallas as pl
from jax.experimental.pallas import tpu as pltpu

_BB = 8
_G = 8
_HM = 512


def _head_body(n, a_ref, ones_ref, wbd_ref, t_ref):
    f32 = jnp.float32
    bf16 = jnp.bfloat16
    ones_bd = ones_ref[...]
    wbd = wbd_ref[...]
    a = a_ref[...]
    rs = jnp.dot(a, ones_bd, preferred_element_type=f32)
    trans = (a.astype(f32) * pl.reciprocal(rs + 1e-12, approx=False)
             ).astype(bf16)
    logits = jnp.dot(wbd, trans, preferred_element_type=f32)
    e = jnp.exp(logits)
    den = jnp.dot(e.astype(bf16), ones_bd, preferred_element_type=f32)
    t_ref[...] = (e * pl.reciprocal(den, approx=False)).astype(bf16)


def _hops_body(n, din, dout, k_hops, bb, g_unroll,
               t_ref, x_ref, ones_ref, lw_ref, lb_ref, o_ref):
    bbn = bb * n
    f32 = jnp.float32
    bf16 = jnp.bfloat16
    ones_bd = ones_ref[...]
    lw = lw_ref[...]
    lb = lb_ref[0:1, :]
    t = t_ref[...]
    xb = x_ref[...].astype(bf16)

    x1s = []
    x2s = []
    for q in range(g_unroll):
        tq = t[q * n:(q + 1) * n, :]
        t_bd = jnp.concatenate([tq] * bb, axis=0) * ones_bd
        xq = xb[q * bbn:(q + 1) * bbn, :]
        x1b = jnp.dot(t_bd, xq, preferred_element_type=f32).astype(bf16)
        x2b = jnp.dot(t_bd, x1b, preferred_element_type=f32).astype(bf16)
        x1s.append(x1b)
        x2s.append(x2b)

    xcat = jnp.concatenate(
        [xb, jnp.concatenate(x1s, axis=0), jnp.concatenate(x2s, axis=0)],
        axis=1)
    acc = jnp.dot(xcat, lw, preferred_element_type=f32)
    acc = (acc + lb) * (1.0 / k_hops)
    o_ref[...] = jnp.maximum(acc, 0.0)


def kernel(X, A, W, lin_w, lin_b):
    f32 = jnp.float32
    bf16 = jnp.bfloat16
    b, n, din = X.shape
    k_hops, _, dout = lin_w.shape
    bb = _BB
    bbn = bb * n
    c = b // bb
    g_unroll = _G
    s = c // g_unroll
    hm = min(_HM, c * n)
    hs = (c * n) // hm

    X2 = X.reshape(b * n, din)

    A_cmp = (A.astype(bf16)
             .reshape(c, bb, n, n)
             .transpose(0, 2, 1, 3)
             .reshape(c * n, bbn))

    ones_bd = jnp.kron(jnp.eye(bb, dtype=bf16), jnp.ones((n, n), bf16))
    w_bd = jnp.kron(jnp.eye(hm // n, dtype=f32),
                    W.reshape(n, n).astype(f32)).astype(bf16)

    lw = lin_w.reshape(k_hops * din, dout).astype(bf16)
    lb = jnp.pad(jnp.sum(lin_b.astype(f32), axis=0, keepdims=True),
                 ((0, 7), (0, 0)))

    t_cmp = pl.pallas_call(
        functools.partial(_head_body, n),
        out_shape=jax.ShapeDtypeStruct((c * n, bbn), bf16),
        grid=(hs,),
        in_specs=[
            pl.BlockSpec((hm, bbn), lambda i: (i, 0)),
            pl.BlockSpec((bbn, bbn), lambda i: (0, 0)),
            pl.BlockSpec((hm, hm), lambda i: (0, 0)),
        ],
        out_specs=pl.BlockSpec((hm, bbn), lambda i: (i, 0)),
        compiler_params=pltpu.CompilerParams(
            dimension_semantics=("parallel",)),
    )(A_cmp, ones_bd, w_bd)

    body = functools.partial(_hops_body, n, din, dout, k_hops, bb, g_unroll)
    out2 = pl.pallas_call(
        body,
        out_shape=jax.ShapeDtypeStruct((b * n, dout), f32),
        grid=(s,),
        in_specs=[
            pl.BlockSpec((g_unroll * n, bbn), lambda i: (i, 0)),
            pl.BlockSpec((g_unroll * bbn, din), lambda i: (i, 0)),
            pl.BlockSpec((bbn, bbn), lambda i: (0, 0)),
            pl.BlockSpec((k_hops * din, dout), lambda i: (0, 0)),
            pl.BlockSpec((8, dout), lambda i: (0, 0)),
        ],
        out_specs=pl.BlockSpec((g_unroll * bbn, dout), lambda i: (i, 0)),
        compiler_params=pltpu.CompilerParams(
            dimension_semantics=("parallel",)),
    )(t_cmp, X2, ones_bd, lw, lb)
    return out2.reshape(b, n, dout)


# speedup vs baseline: 5.0887x; 1.2196x over previous
import functools

import jax
import jax.numpy as jnp
from jax.experimental import pallas as pl
from jax.experimental.pallas import tpu as pltpu

_BB = 8
_G = 32


def _diff_conv_body(n, din, dout, k_hops, bb, g_unroll,
                    a_ref, x_ref, ones_ref, wbd_ref, lw_ref, lb_ref,
                    o_ref, xcat_ref):
    bbn = bb * n
    f32 = jnp.float32
    bf16 = jnp.bfloat16
    ones_bd = ones_ref[...]
    wbd = wbd_ref[...]
    lw = lw_ref[...]
    lb = lb_ref[0:1, :]

    a = a_ref[...]
    rs = jnp.dot(a, ones_bd, preferred_element_type=f32)
    trans = (a.astype(f32) * pl.reciprocal(rs + 1e-12, approx=False)
             ).astype(bf16)
    logits = jnp.dot(wbd, trans, preferred_element_type=f32)
    e = jnp.exp(logits)
    den = jnp.dot(e.astype(bf16), ones_bd, preferred_element_type=f32)
    t = (e * pl.reciprocal(den, approx=False)).astype(bf16)

    xb = x_ref[...].astype(bf16)
    xcat_ref[:, 0:din] = xb
    for q in range(g_unroll):
        tq = t[q * n:(q + 1) * n, :]
        t_bd = jnp.concatenate([tq] * bb, axis=0) * ones_bd
        xq = xb[q * bbn:(q + 1) * bbn, :]
        x1b = jnp.dot(t_bd, xq, preferred_element_type=f32).astype(bf16)
        x2b = jnp.dot(t_bd, x1b, preferred_element_type=f32).astype(bf16)
        xcat_ref[q * bbn:(q + 1) * bbn, din:2 * din] = x1b
        xcat_ref[q * bbn:(q + 1) * bbn, 2 * din:3 * din] = x2b

    acc = jnp.dot(xcat_ref[...], lw, preferred_element_type=f32)
    acc = (acc + lb) * (1.0 / k_hops)
    o_ref[...] = jnp.maximum(acc, 0.0)


def kernel(X, A, W, lin_w, lin_b):
    f32 = jnp.float32
    bf16 = jnp.bfloat16
    b, n, din = X.shape
    k_hops, _, dout = lin_w.shape
    bb = _BB
    bbn = bb * n
    c = b // bb
    g_unroll = min(_G, c)
    s = c // g_unroll

    X2 = X.reshape(b * n, din)

    A_cmp = (A.astype(bf16)
             .reshape(c, bb, n, n)
             .transpose(0, 2, 1, 3)
             .reshape(c * n, bbn))

    ones_bd = jnp.kron(jnp.eye(bb, dtype=bf16), jnp.ones((n, n), bf16))
    w_bd = jnp.kron(jnp.eye(g_unroll, dtype=f32),
                    W.reshape(n, n).astype(f32)).astype(bf16)

    lw = lin_w.reshape(k_hops * din, dout).astype(bf16)
    lb = jnp.pad(jnp.sum(lin_b.astype(f32), axis=0, keepdims=True),
                 ((0, 7), (0, 0)))

    body = functools.partial(_diff_conv_body, n, din, dout, k_hops, bb,
                             g_unroll)
    out2 = pl.pallas_call(
        body,
        out_shape=jax.ShapeDtypeStruct((b * n, dout), f32),
        grid=(s,),
        in_specs=[
            pl.BlockSpec((g_unroll * n, bbn), lambda i: (i, 0)),
            pl.BlockSpec((g_unroll * bbn, din), lambda i: (i, 0)),
            pl.BlockSpec((bbn, bbn), lambda i: (0, 0)),
            pl.BlockSpec((g_unroll * n, g_unroll * n), lambda i: (0, 0)),
            pl.BlockSpec((k_hops * din, dout), lambda i: (0, 0)),
            pl.BlockSpec((8, dout), lambda i: (0, 0)),
        ],
        out_specs=pl.BlockSpec((g_unroll * bbn, dout), lambda i: (i, 0)),
        scratch_shapes=[
            pltpu.VMEM((g_unroll * bbn, k_hops * din), bf16)],
        compiler_params=pltpu.CompilerParams(
            dimension_semantics=("parallel",)),
    )(A_cmp, X2, ones_bd, w_bd, lw, lb)
    return out2.reshape(b, n, dout)


# speedup vs baseline: 5.4797x; 1.0768x over previous
import functools

import jax
import jax.numpy as jnp
from jax.experimental import pallas as pl
from jax.experimental.pallas import tpu as pltpu

_BB = 8
_G = 16


def _diff_conv_body(n, din, dout, k_hops, bb, g_unroll,
                    a_ref, x_ref, ones_ref, wbd_ref, lw_ref, lb_ref,
                    o_ref, xcat_ref):
    bbn = bb * n
    f32 = jnp.float32
    bf16 = jnp.bfloat16
    ones_bd = ones_ref[...]
    wbd = wbd_ref[...]
    lw = lw_ref[...]
    lb = lb_ref[0:1, :]

    a = a_ref[...]
    rs = jnp.dot(a, ones_bd, preferred_element_type=f32)
    trans = (a.astype(f32) * pl.reciprocal(rs + 1e-12, approx=False)
             ).astype(bf16)
    logits = jnp.dot(wbd, trans, preferred_element_type=f32)
    e = jnp.exp(logits)
    den = jnp.dot(e.astype(bf16), ones_bd, preferred_element_type=f32)
    t = (e * pl.reciprocal(den, approx=False)).astype(bf16)

    xb = x_ref[...].astype(bf16)
    xcat_ref[:, 0:din] = xb
    for q in range(g_unroll):
        tq = t[q * n:(q + 1) * n, :]
        t_bd = jnp.concatenate([tq] * bb, axis=0) * ones_bd
        xq = xb[q * bbn:(q + 1) * bbn, :]
        x1b = jnp.dot(t_bd, xq, preferred_element_type=f32).astype(bf16)
        x2b = jnp.dot(t_bd, x1b, preferred_element_type=f32).astype(bf16)
        xcat_ref[q * bbn:(q + 1) * bbn, din:2 * din] = x1b
        xcat_ref[q * bbn:(q + 1) * bbn, 2 * din:3 * din] = x2b

    acc = jnp.dot(xcat_ref[...], lw, preferred_element_type=f32)
    acc = (acc + lb) * (1.0 / k_hops)
    o_ref[...] = jnp.maximum(acc, 0.0)


def kernel(X, A, W, lin_w, lin_b):
    f32 = jnp.float32
    bf16 = jnp.bfloat16
    b, n, din = X.shape
    k_hops, _, dout = lin_w.shape
    bb = _BB
    bbn = bb * n
    c = b // bb
    g_unroll = min(_G, c)
    s = c // g_unroll

    X2 = X.reshape(b * n, din)

    A_cmp = (A.astype(bf16)
             .reshape(c, bb, n, n)
             .transpose(0, 2, 1, 3)
             .reshape(c * n, bbn))

    ones_bd = jnp.kron(jnp.eye(bb, dtype=bf16), jnp.ones((n, n), bf16))
    w_bd = jnp.kron(jnp.eye(g_unroll, dtype=f32),
                    W.reshape(n, n).astype(f32)).astype(bf16)

    lw = lin_w.reshape(k_hops * din, dout).astype(bf16)
    lb = jnp.pad(jnp.sum(lin_b.astype(f32), axis=0, keepdims=True),
                 ((0, 7), (0, 0)))

    body = functools.partial(_diff_conv_body, n, din, dout, k_hops, bb,
                             g_unroll)
    out2 = pl.pallas_call(
        body,
        out_shape=jax.ShapeDtypeStruct((b * n, dout), f32),
        grid=(s,),
        in_specs=[
            pl.BlockSpec((g_unroll * n, bbn), lambda i: (i, 0)),
            pl.BlockSpec((g_unroll * bbn, din), lambda i: (i, 0)),
            pl.BlockSpec((bbn, bbn), lambda i: (0, 0)),
            pl.BlockSpec((g_unroll * n, g_unroll * n), lambda i: (0, 0)),
            pl.BlockSpec((k_hops * din, dout), lambda i: (0, 0)),
            pl.BlockSpec((8, dout), lambda i: (0, 0)),
        ],
        out_specs=pl.BlockSpec((g_unroll * bbn, dout), lambda i: (i, 0)),
        scratch_shapes=[
            pltpu.VMEM((g_unroll * bbn, k_hops * din), bf16)],
        compiler_params=pltpu.CompilerParams(
            dimension_semantics=("parallel",)),
    )(A_cmp, X2, ones_bd, w_bd, lw, lb)
    return out2.reshape(b, n, dout)
